# Initial kernel scaffold; baseline (speedup 1.0000x reference)
import jax
import jax.numpy as jnp
from jax.experimental import pallas as pl


def kernel(x, token_table, pos_table):
    raise NotImplementedError("write your pallas kernel here")


# speedup vs baseline: 2.7855x; 2.7855x over previous
import functools

import jax
import jax.numpy as jnp
from jax import lax
from jax.experimental import pallas as pl
from jax.experimental.pallas import tpu as pltpu
from jax.experimental.pallas import tpu_sc as plsc

VOCAB = 100000
SEQ = 200
DIM = 64
BATCH = 4096

_NC = 2
_NS = 16
_NW = _NC * _NS
_BPW = BATCH // _NW
_H = SEQ // 2


def _tpe_body(x_hbm, tbl_hbm, pos_hbm, out_hbm, idx_v, rows_v, pos_v, sem):
    wid = lax.axis_index("s") * _NC + lax.axis_index("c")

    pltpu.sync_copy(pos_hbm, pos_v)

    @pl.loop(0, _BPW)
    def _row(i):
        b = wid * _BPW + i
        pltpu.sync_copy(x_hbm.at[b], idx_v)
        c0 = pltpu.async_copy(tbl_hbm.at[idx_v.at[0]], rows_v.at[0], sem)
        c1 = pltpu.async_copy(tbl_hbm.at[idx_v.at[1]], rows_v.at[1], sem)
        c0.wait()
        c1.wait()

        @pl.loop(0, _H)
        def _pos(s):
            for k in range(2):
                for j in range(DIM // 16):
                    sl = pl.ds(j * 16, 16)
                    rows_v[k, s, sl] = rows_v[k, s, sl] + pos_v[k, s, sl]

        pltpu.sync_copy(rows_v, out_hbm.at[b])


@jax.jit
def _tpe(x3, token_table, pos3):
    f = functools.partial(
        pl.kernel,
        out_type=jax.ShapeDtypeStruct((BATCH, 2, _H, DIM), jnp.float32),
        mesh=plsc.VectorSubcoreMesh(core_axis_name="c", subcore_axis_name="s"),
        scratch_types=[
            pltpu.VMEM((2, _H), jnp.int32),
            pltpu.VMEM((2, _H, DIM), jnp.float32),
            pltpu.VMEM((2, _H, DIM), jnp.float32),
            pltpu.SemaphoreType.DMA,
        ],
        compiler_params=pltpu.CompilerParams(use_tc_tiling_on_sc=False),
    )(_tpe_body)
    return f(x3, token_table, pos3)


def kernel(x, token_table, pos_table):
    x3 = x.reshape(BATCH, 2, _H).astype(jnp.int32)
    pos3 = pos_table.reshape(2, _H, DIM)
    out = _tpe(x3, token_table, pos3)
    return out.reshape(BATCH, SEQ, DIM)


# speedup vs baseline: 3.6222x; 1.3004x over previous
import functools

import jax
import jax.numpy as jnp
from jax import lax
from jax.experimental import pallas as pl
from jax.experimental.pallas import tpu as pltpu
from jax.experimental.pallas import tpu_sc as plsc

VOCAB = 100000
SEQ = 200
DIM = 64
BATCH = 4096

_NC = 2
_NS = 16
_NW = _NC * _NS
_BPW = BATCH // _NW
_H = SEQ // 2
_CH = 2
_NCHUNK = _BPW // _CH


def _tpe_body(x_hbm, tbl_hbm, pos_hbm, out_hbm,
              idx_v, pos_v, rows0, rows1, g0, g1, o0, o1):
    wid = lax.axis_index("s") * _NC + lax.axis_index("c")
    b0 = wid * _BPW

    rows = (rows0, rows1)
    gsem = (g0, g1)
    osem = (o0, o1)

    pltpu.sync_copy(pos_hbm, pos_v)
    pltpu.sync_copy(x_hbm.at[pl.ds(b0, _BPW)], idx_v)

    def gather(i, par):
        cps = []
        for c in range(_CH):
            for k in range(2):
                cps.append(pltpu.make_async_copy(
                    tbl_hbm.at[idx_v.at[i * _CH + c, k]],
                    rows[par].at[c, k], gsem[par]))
        return cps

    def outcopy(i, par):
        return pltpu.make_async_copy(
            rows[par], out_hbm.at[pl.ds(b0 + i * _CH, _CH)], osem[par])

    for cp in gather(0, 0):
        cp.start()

    @pl.loop(0, _NCHUNK, step=2)
    def _chunk(g):
        for par in range(2):
            i = g + par
            nxt = 1 - par

            @pl.when(i + 1 < _NCHUNK)
            def _():
                @pl.when(i >= 1)
                def _():
                    outcopy(i - 1, nxt).wait()
                for cp in gather(i + 1, nxt):
                    cp.start()

            for cp in gather(i, par):
                cp.wait()

            @pl.loop(0, _H)
            def _pos(s):
                for k in range(2):
                    for j in range(DIM // 16):
                        sl = pl.ds(j * 16, 16)
                        p = pos_v[k, s, sl]
                        for c in range(_CH):
                            rows[par][c, k, s, sl] = rows[par][c, k, s, sl] + p

            outcopy(i, par).start()

    outcopy(_NCHUNK - 2, 0).wait()
    outcopy(_NCHUNK - 1, 1).wait()


@jax.jit
def _tpe(x3, token_table, pos3):
    f = functools.partial(
        pl.kernel,
        out_type=jax.ShapeDtypeStruct((BATCH, 2, _H, DIM), jnp.float32),
        mesh=plsc.VectorSubcoreMesh(core_axis_name="c", subcore_axis_name="s"),
        scratch_types=[
            pltpu.VMEM((_BPW, 2, _H), jnp.int32),
            pltpu.VMEM((2, _H, DIM), jnp.float32),
            pltpu.VMEM((_CH, 2, _H, DIM), jnp.float32),
            pltpu.VMEM((_CH, 2, _H, DIM), jnp.float32),
            pltpu.SemaphoreType.DMA,
            pltpu.SemaphoreType.DMA,
            pltpu.SemaphoreType.DMA,
            pltpu.SemaphoreType.DMA,
        ],
        compiler_params=pltpu.CompilerParams(use_tc_tiling_on_sc=False),
    )(_tpe_body)
    return f(x3, token_table, pos3)


def kernel(x, token_table, pos_table):
    x3 = x.reshape(BATCH, 2, _H).astype(jnp.int32)
    pos3 = pos_table.reshape(2, _H, DIM)
    out = _tpe(x3, token_table, pos3)
    return out.reshape(BATCH, SEQ, DIM)
